# baseline (speedup 1.0000x reference)
import functools

import jax
import jax.numpy as jnp
from jax import lax
from jax.experimental import pallas as pl
from jax.experimental.pallas import tpu as pltpu
from jax.experimental.pallas import tpu_sc as plsc

NUM_HEADS = 4
ROWS_PER_TABLE = 100000
DIM = 64
LANES = 16
NC, NS = 2, 16
NW = NC * NS
CHUNK = 64
IDX_ROW = 128
GPC = 2
SUPER_ROWS = 8
OUT_ROWS = CHUNK * DIM // 128


def _mhe_sc(idx2d, tab2, B):
    ntok_w = B // NW
    nchunks = ntok_w // CHUNK
    nouter = nchunks // 8
    mesh = plsc.VectorSubcoreMesh(core_axis_name="c", subcore_axis_name="s")

    @functools.partial(
        pl.kernel,
        out_type=jax.ShapeDtypeStruct((B * DIM // 128, 128), jnp.float32),
        mesh=mesh,
        scratch_types=[
            [pltpu.VMEM((SUPER_ROWS, IDX_ROW), jnp.int32) for _ in range(2)],
            [pltpu.VMEM((SUPER_ROWS, IDX_ROW), jnp.int32) for _ in range(2)],
            [pltpu.VMEM((SUPER_ROWS, IDX_ROW), jnp.int32) for _ in range(2)],
            [pltpu.VMEM((CHUNK * NUM_HEADS, 128), jnp.float32)
             for _ in range(2)],
            [pltpu.VMEM((OUT_ROWS, 128), jnp.float32) for _ in range(2)],
            [pltpu.SemaphoreType.DMA for _ in range(4)],
        ],
    )
    def k(idx_hbm, tab_hbm, out_hbm, raw, gidx, pcol, buf, outv, sems):
        gsem = sems[:2]
        osem = sems[2:]
        wid = lax.axis_index("s") * NC + lax.axis_index("c")
        offs = (lax.iota(jnp.int32, LANES) & (NUM_HEADS - 1)) * ROWS_PER_TABLE
        chunk_base = wid * nchunks

        def stage(sm, half):
            src0 = pl.multiple_of(sm * SUPER_ROWS, SUPER_ROWS)
            pltpu.sync_copy(idx_hbm.at[pl.ds(src0, SUPER_ROWS)], raw[half])
            for j in range(SUPER_ROWS):
                for kk in range(IDX_ROW // LANES):
                    sl = pl.ds(kk * LANES, LANES)
                    flat = raw[half][j, sl] + offs
                    gidx[half][j, sl] = flat >> 1
                    pcol[half][j, sl] = (flat & 1) * DIM

        def fire(side, half, r0):
            for j in range(GPC):
                pltpu.async_copy(
                    tab_hbm.at[gidx[half].at[r0 + j]],
                    buf[side].at[pl.ds(j * IDX_ROW, IDX_ROW)],
                    gsem[side],
                )

        def drain(side, half, r0):
            for j in range(GPC):
                pltpu.make_async_copy(
                    tab_hbm.at[gidx[half].at[r0 + j]],
                    buf[side].at[pl.ds(j * IDX_ROW, IDX_ROW)],
                    gsem[side],
                ).wait()

        def consume(side, half, r0, c, wait_prev):
            tok0 = (chunk_base + c) * CHUNK
            orow0 = pl.multiple_of(tok0 * DIM // 128, OUT_ROWS)
            out_v = outv[side]
            buf_v = buf[side]

            @pl.when(wait_prev)
            def _():
                pltpu.make_async_copy(
                    out_v, out_hbm.at[pl.ds(orow0, OUT_ROWS)],
                    osem[side]).wait()

            def acc_body(gg, c2):
                prow = r0 + gg // 8
                pv = pcol[half][prow, pl.ds((gg % 8) * LANES, LANES)]
                for tt in range(4):
                    t = gg * 4 + tt
                    r = gg * LANES + 4 * tt
                    orow = t // 2
                    ocol = (t % 2) * DIM
                    p0 = pv[4 * tt + 0]
                    p1 = pv[4 * tt + 1]
                    p2 = pv[4 * tt + 2]
                    p3 = pv[4 * tt + 3]
                    for d in range(DIM // LANES):
                        o = d * LANES
                        out_v[orow, pl.ds(ocol + o, LANES)] = (
                            buf_v[r + 0, pl.ds(p0 + o, LANES)]
                            + buf_v[r + 1, pl.ds(p1 + o, LANES)]
                        ) + (
                            buf_v[r + 2, pl.ds(p2 + o, LANES)]
                            + buf_v[r + 3, pl.ds(p3 + o, LANES)]
                        )
                return c2

            lax.fori_loop(0, CHUNK // 4, acc_body, 0)
            pltpu.async_copy(
                out_v, out_hbm.at[pl.ds(orow0, OUT_ROWS)], osem[side])

        super_base = chunk_base // 4
        stage(super_base, 0)
        fire(0, 0, 0)

        H_C = (0, 0, 1, 1)
        R_C = (0, 4, 0, 4)
        H_N = (0, 1, 1, 0)
        R_N = (4, 0, 4, 0)

        def body(q, carry):
            for s in range(4):
                g = 4 * q + s
                c0 = 2 * g
                c1 = c0 + 1
                hc, rc = H_C[s], R_C[s]
                fire(1, hc, rc + 2)
                drain(0, hc, rc)
                consume(0, hc, rc, c0, jnp.logical_or(q > 0, s > 0))

                if s % 2 == 1:
                    sm = super_base + (c0 + 2) // 4

                    @pl.when(c0 + 2 < nchunks)
                    def _(sm=sm, s=s):
                        stage(sm, H_N[s])

                @pl.when(c0 + 2 < nchunks)
                def _(s=s):
                    fire(0, H_N[s], R_N[s])

                drain(1, hc, rc + 2)
                consume(1, hc, rc + 2, c1, jnp.logical_or(q > 0, s > 0))
            return carry

        lax.fori_loop(0, nouter, body, 0)
        obase = pl.multiple_of(chunk_base * CHUNK * DIM // 128, OUT_ROWS)
        for side in range(2):
            pltpu.make_async_copy(
                outv[side], out_hbm.at[pl.ds(obase, OUT_ROWS)],
                osem[side]).wait()

    return k(idx2d, tab2)


def kernel(input, tables):
    bd, t, h = input.shape
    B = bd * t
    idx2d = input.astype(jnp.int32).reshape(B * h // IDX_ROW, IDX_ROW)
    tab2 = tables.reshape(h * ROWS_PER_TABLE * DIM // 128, 128)
    out = _mhe_sc(idx2d, tab2, B)
    return out.reshape(bd, t, DIM)


# speedup vs baseline: 1.0320x; 1.0320x over previous
import functools

import jax
import jax.numpy as jnp
from jax import lax
from jax.experimental import pallas as pl
from jax.experimental.pallas import tpu as pltpu
from jax.experimental.pallas import tpu_sc as plsc

NUM_HEADS = 4
ROWS_PER_TABLE = 100000
DIM = 64
LANES = 16
NC, NS = 2, 16
NW = NC * NS
CHUNK = 128
IDX_ROW = 128
ROWS_PER_CHUNK = CHUNK * NUM_HEADS // IDX_ROW


def _mhe_sc(idx2d, tab_flat, B):
    ntok_w = B // NW
    nchunks = ntok_w // CHUNK
    mesh = plsc.VectorSubcoreMesh(core_axis_name="c", subcore_axis_name="s")

    @functools.partial(
        pl.kernel,
        out_type=jax.ShapeDtypeStruct((B * DIM // 128, 128), jnp.float32),
        mesh=mesh,
        compiler_params=pltpu.CompilerParams(use_tc_tiling_on_sc=False),
        scratch_types=[
            pltpu.VMEM((ROWS_PER_CHUNK, IDX_ROW), jnp.int32),
            pltpu.VMEM((ROWS_PER_CHUNK, IDX_ROW), jnp.int32),
            pltpu.VMEM((CHUNK * NUM_HEADS, DIM), jnp.float32),
            pltpu.VMEM((CHUNK * NUM_HEADS, DIM), jnp.float32),
            pltpu.VMEM((CHUNK * DIM // 128, 128), jnp.float32),
            pltpu.VMEM((CHUNK * DIM // 128, 128), jnp.float32),
            pltpu.SemaphoreType.DMA,
            pltpu.SemaphoreType.DMA,
            pltpu.SemaphoreType.DMA,
            pltpu.SemaphoreType.DMA,
        ],
    )
    def k(idx_hbm, tab_hbm, out_hbm, idx_a, idx_b, buf_a, buf_b, out_a,
          out_b, gsem_a, gsem_b, osem_a, osem_b):
        wid = lax.axis_index("s") * NC + lax.axis_index("c")
        offs = (lax.iota(jnp.int32, LANES) & (NUM_HEADS - 1)) * ROWS_PER_TABLE

        def fire(idx_v, buf_v, gsem, tok0):
            row0 = pl.multiple_of(
                tok0 * NUM_HEADS // IDX_ROW, ROWS_PER_CHUNK)
            pltpu.sync_copy(idx_hbm.at[pl.ds(row0, ROWS_PER_CHUNK)], idx_v)
            for j in range(ROWS_PER_CHUNK):
                for kk in range(IDX_ROW // LANES):
                    sl = pl.ds(kk * LANES, LANES)
                    idx_v[j, sl] = idx_v[j, sl] + offs
            for j in range(ROWS_PER_CHUNK):
                pltpu.async_copy(
                    tab_hbm.at[idx_v.at[j]],
                    buf_v.at[pl.ds(j * IDX_ROW, IDX_ROW)],
                    gsem,
                )

        def drain(idx_v, buf_v, gsem):
            for j in range(ROWS_PER_CHUNK):
                pltpu.make_async_copy(
                    tab_hbm.at[idx_v.at[j]],
                    buf_v.at[pl.ds(j * IDX_ROW, IDX_ROW)],
                    gsem,
                ).wait()

        def consume(buf_v, out_v, osem, tok0, wait_prev):
            orow0 = pl.multiple_of(tok0 * DIM // 128, CHUNK * DIM // 128)

            @pl.when(wait_prev)
            def _():
                pltpu.make_async_copy(
                    out_v, out_hbm.at[pl.ds(orow0, CHUNK * DIM // 128)],
                    osem).wait()

            def acc_body(tp, c2):
                orow = tp
                for u in range(2):
                    t = tp * 2 + u
                    r = t * NUM_HEADS
                    for d in range(DIM // LANES):
                        sl = pl.ds(d * LANES, LANES)
                        osl = pl.ds(u * DIM + d * LANES, LANES)
                        out_v[orow, osl] = (
                            buf_v[r, sl] + buf_v[r + 1, sl]
                        ) + (buf_v[r + 2, sl] + buf_v[r + 3, sl])
                return c2

            lax.fori_loop(0, CHUNK // 2, acc_body, 0)
            pltpu.async_copy(
                out_v, out_hbm.at[pl.ds(orow0, CHUNK * DIM // 128)], osem)

        tok_base = wid * ntok_w
        fire(idx_a, buf_a, gsem_a, pl.multiple_of(tok_base, CHUNK))

        def body(g, carry):
            c0 = pl.multiple_of(tok_base + 2 * g * CHUNK, CHUNK)
            c1 = pl.multiple_of(c0 + CHUNK, CHUNK)
            fire(idx_b, buf_b, gsem_b, c1)
            drain(idx_a, buf_a, gsem_a)
            consume(buf_a, out_a, osem_a, c0, g > 0)

            @pl.when(2 * g + 2 < nchunks)
            def _():
                fire(idx_a, buf_a, gsem_a, pl.multiple_of(c0 + 2 * CHUNK,
                                                          CHUNK))

            drain(idx_b, buf_b, gsem_b)
            consume(buf_b, out_b, osem_b, c1, g > 0)
            return carry

        lax.fori_loop(0, nchunks // 2, body, 0)
        obase = pl.multiple_of(tok_base * DIM // 128, CHUNK * DIM // 128)
        pltpu.make_async_copy(
            out_a, out_hbm.at[pl.ds(obase, CHUNK * DIM // 128)],
            osem_a).wait()
        pltpu.make_async_copy(
            out_b, out_hbm.at[pl.ds(obase, CHUNK * DIM // 128)],
            osem_b).wait()

    return k(idx2d, tab_flat)


def kernel(input, tables):
    bd, t, h = input.shape
    B = bd * t
    idx2d = input.astype(jnp.int32).reshape(B * h // IDX_ROW, IDX_ROW)
    tab_flat = tables.reshape(h * ROWS_PER_TABLE, DIM)
    out = _mhe_sc(idx2d, tab_flat, B)
    return out.reshape(bd, t, DIM)


# speedup vs baseline: 1.0802x; 1.0467x over previous
import functools

import jax
import jax.numpy as jnp
from jax import lax
from jax.experimental import pallas as pl
from jax.experimental.pallas import tpu as pltpu
from jax.experimental.pallas import tpu_sc as plsc

NUM_HEADS = 4
ROWS_PER_TABLE = 100000
DIM = 64
LANES = 16
NC, NS = 2, 16
NW = NC * NS
CHUNK = 128
IDX_ROW = 128
ROWS_PER_CHUNK = CHUNK * NUM_HEADS // IDX_ROW


def _mhe_sc(idx2d, tab_flat, B):
    ntok_w = B // NW
    nchunks = ntok_w // CHUNK
    mesh = plsc.VectorSubcoreMesh(core_axis_name="c", subcore_axis_name="s")

    @functools.partial(
        pl.kernel,
        out_type=jax.ShapeDtypeStruct((B * DIM // 128, 128), jnp.float32),
        mesh=mesh,
        compiler_params=pltpu.CompilerParams(use_tc_tiling_on_sc=False),
        scratch_types=[
            pltpu.VMEM((ROWS_PER_CHUNK, IDX_ROW), jnp.int32),
            pltpu.VMEM((ROWS_PER_CHUNK, IDX_ROW), jnp.int32),
            pltpu.VMEM((CHUNK * NUM_HEADS, DIM), jnp.float32),
            pltpu.VMEM((CHUNK * NUM_HEADS, DIM), jnp.float32),
            pltpu.VMEM((CHUNK * DIM // 128, 128), jnp.float32),
            pltpu.VMEM((CHUNK * DIM // 128, 128), jnp.float32),
            pltpu.SemaphoreType.DMA,
            pltpu.SemaphoreType.DMA,
            pltpu.SemaphoreType.DMA,
            pltpu.SemaphoreType.DMA,
            pltpu.SemaphoreType.DMA,
            pltpu.SemaphoreType.DMA,
        ],
    )
    def k(idx_hbm, tab_hbm, out_hbm, idx_a, idx_b, buf_a, buf_b, out_a,
          out_b, gsem_a, gsem_b, osem_a, osem_b, isem_a, isem_b):
        wid = lax.axis_index("s") * NC + lax.axis_index("c")
        offs = (lax.iota(jnp.int32, LANES) & (NUM_HEADS - 1)) * ROWS_PER_TABLE

        def _idx_rows(tok0):
            return pl.multiple_of(
                tok0 * NUM_HEADS // IDX_ROW, ROWS_PER_CHUNK)

        def stage_idx(idx_v, isem, tok0):
            pltpu.async_copy(
                idx_hbm.at[pl.ds(_idx_rows(tok0), ROWS_PER_CHUNK)],
                idx_v, isem)

        def fire(idx_v, buf_v, gsem, isem, tok0):
            pltpu.make_async_copy(
                idx_hbm.at[pl.ds(_idx_rows(tok0), ROWS_PER_CHUNK)],
                idx_v, isem).wait()
            for j in range(ROWS_PER_CHUNK):
                for kk in range(IDX_ROW // LANES):
                    sl = pl.ds(kk * LANES, LANES)
                    idx_v[j, sl] = idx_v[j, sl] + offs
            for j in range(ROWS_PER_CHUNK):
                pltpu.async_copy(
                    tab_hbm.at[idx_v.at[j]],
                    buf_v.at[pl.ds(j * IDX_ROW, IDX_ROW)],
                    gsem,
                )

        def drain(idx_v, buf_v, gsem):
            for j in range(ROWS_PER_CHUNK):
                pltpu.make_async_copy(
                    tab_hbm.at[idx_v.at[j]],
                    buf_v.at[pl.ds(j * IDX_ROW, IDX_ROW)],
                    gsem,
                ).wait()

        def consume(buf_v, out_v, osem, tok0, wait_prev):
            orow0 = pl.multiple_of(tok0 * DIM // 128, CHUNK * DIM // 128)

            @pl.when(wait_prev)
            def _():
                pltpu.make_async_copy(
                    out_v, out_hbm.at[pl.ds(orow0, CHUNK * DIM // 128)],
                    osem).wait()

            def acc_body(tp, c2):
                orow = tp
                for u in range(2):
                    t = tp * 2 + u
                    r = t * NUM_HEADS
                    for d in range(DIM // LANES):
                        sl = pl.ds(d * LANES, LANES)
                        osl = pl.ds(u * DIM + d * LANES, LANES)
                        out_v[orow, osl] = (
                            buf_v[r, sl] + buf_v[r + 1, sl]
                        ) + (buf_v[r + 2, sl] + buf_v[r + 3, sl])
                return c2

            lax.fori_loop(0, CHUNK // 2, acc_body, 0)
            pltpu.async_copy(
                out_v, out_hbm.at[pl.ds(orow0, CHUNK * DIM // 128)], osem)

        tok_base = wid * ntok_w
        stage_idx(idx_a, isem_a, pl.multiple_of(tok_base, CHUNK))
        fire(idx_a, buf_a, gsem_a, isem_a, pl.multiple_of(tok_base, CHUNK))
        stage_idx(idx_b, isem_b, pl.multiple_of(tok_base + CHUNK, CHUNK))

        def body(g, carry):
            c0 = pl.multiple_of(tok_base + 2 * g * CHUNK, CHUNK)
            c1 = pl.multiple_of(c0 + CHUNK, CHUNK)
            c2 = pl.multiple_of(c0 + 2 * CHUNK, CHUNK)
            c3 = pl.multiple_of(c0 + 3 * CHUNK, CHUNK)
            fire(idx_b, buf_b, gsem_b, isem_b, c1)
            drain(idx_a, buf_a, gsem_a)

            @pl.when(2 * g + 2 < nchunks)
            def _():
                stage_idx(idx_a, isem_a, c2)

            consume(buf_a, out_a, osem_a, c0, g > 0)

            @pl.when(2 * g + 2 < nchunks)
            def _():
                fire(idx_a, buf_a, gsem_a, isem_a, c2)

            drain(idx_b, buf_b, gsem_b)

            @pl.when(2 * g + 3 < nchunks)
            def _():
                stage_idx(idx_b, isem_b, c3)

            consume(buf_b, out_b, osem_b, c1, g > 0)
            return carry

        lax.fori_loop(0, nchunks // 2, body, 0)
        obase = pl.multiple_of(tok_base * DIM // 128, CHUNK * DIM // 128)
        pltpu.make_async_copy(
            out_a, out_hbm.at[pl.ds(obase, CHUNK * DIM // 128)],
            osem_a).wait()
        pltpu.make_async_copy(
            out_b, out_hbm.at[pl.ds(obase, CHUNK * DIM // 128)],
            osem_b).wait()

    return k(idx2d, tab_flat)


def kernel(input, tables):
    bd, t, h = input.shape
    B = bd * t
    idx2d = input.astype(jnp.int32).reshape(B * h // IDX_ROW, IDX_ROW)
    tab_flat = tables.reshape(h * ROWS_PER_TABLE, DIM)
    out = _mhe_sc(idx2d, tab_flat, B)
    return out.reshape(bd, t, DIM)
